# baseline (speedup 1.0000x reference)
import functools

import jax
import jax.numpy as jnp
from jax import lax
from jax.experimental import pallas as pl
from jax.experimental.pallas import tpu as pltpu
from jax.experimental.pallas import tpu_sc as plsc

_T = 32768
_K = _T // 16 + 1
_ROWS = 128
_NV = _T // 16
_NB = 1024
_NW = 32

_TC_ROWS = 96
_SC_ROWS = _ROWS - _TC_ROWS
_SC_RPW = _SC_ROWS // _NW

_HI_BITS = 0x3F800000
_SEARCH_ITERS = 30
_W_ACC = 2048

_mesh = plsc.VectorSubcoreMesh(core_axis_name="c", subcore_axis_name="s")


def _scan_hist(hist_c, hist_s, r):
    iota = lax.iota(jnp.int32, 16)

    def coarse(i, carry):
        c_hi, s_hi, jsel, cbase, sbase = carry
        j = (_NB // 16 - 1) - i
        tot = jnp.sum(hist_c[pl.ds(j * 16, 16)])
        tot_s = jnp.sum(hist_s[pl.ds(j * 16, 16)])
        new_c = c_hi + tot
        hit = jnp.logical_and(c_hi < r, new_c >= r)
        jsel = jnp.where(hit, j, jsel)
        cbase = jnp.where(hit, c_hi, cbase)
        sbase = jnp.where(hit, s_hi, sbase)
        return new_c, s_hi + tot_s, jsel, cbase, sbase

    _, _, jsel, cbase, sbase = lax.fori_loop(
        0, _NB // 16, coarse,
        (jnp.int32(0), jnp.float32(0.0), jnp.int32(0), jnp.int32(0),
         jnp.float32(0.0)))

    c = hist_c[pl.ds(jsel * 16, 16)]
    s = hist_s[pl.ds(jsel * 16, 16)]
    cs = jnp.cumsum(c)
    css = jnp.cumsum(s)
    tot = jnp.max(cs)
    tot_s = jnp.sum(s)
    suf_next = cbase + tot - cs
    cond = jnp.logical_and(suf_next + c >= r, suf_next < r)
    bsel = jnp.max(jnp.where(cond, jsel * 16 + iota, -1))
    c_above = jnp.max(jnp.where(cond, suf_next, -1))
    s_above = jnp.max(jnp.where(cond, sbase + tot_s - css, -1.0))
    return bsel, c_above, s_above


@functools.partial(
    pl.kernel,
    mesh=_mesh,
    out_type=jax.ShapeDtypeStruct((_NW, 16), jnp.float32),
    compiler_params=pltpu.CompilerParams(needs_layout_passes=False),
    scratch_types=[
        pltpu.VMEM((_T,), jnp.float32),
        pltpu.VMEM((_T,), jnp.float32),
        pltpu.VMEM((_NB,), jnp.int32),
        pltpu.VMEM((_NB,), jnp.float32),
        pltpu.VMEM((16,), jnp.float32),
        pltpu.SemaphoreType.DMA,
    ],
)
def _sc_topk(frame_hbm, out_hbm, row0_v, row1_v, hist_c, hist_s, out_v, sem):
    wid = lax.axis_index("s") * 2 + lax.axis_index("c")
    ones16 = jnp.ones((16,), jnp.int32)
    iota = lax.iota(jnp.int32, 16)
    outvec = jnp.zeros((16,), jnp.float32)
    bufs = (row0_v, row1_v)
    base = _TC_ROWS + wid * _SC_RPW

    pltpu.async_copy(frame_hbm.at[base], bufs[0], sem).wait()

    for j in range(_SC_RPW):
        row_v = bufs[j % 2]
        if j + 1 < _SC_RPW:
            nxt = pltpu.async_copy(frame_hbm.at[base + j + 1],
                                   bufs[(j + 1) % 2], sem)

        def hist_pass(shift, prefix_shift, prefix):
            z_i = jnp.zeros((16,), jnp.int32)
            z_f = jnp.zeros((16,), jnp.float32)

            @plsc.parallel_loop(0, _NB // 16, 1, unroll=8)
            def _(i):
                hist_c[pl.ds(i * 16, 16)] = z_i
                hist_s[pl.ds(i * 16, 16)] = z_f

            @plsc.parallel_loop(0, _NV, 1, unroll=8)
            def _(i):
                v = row_v[pl.ds(i * 16, 16)]
                bits = lax.bitcast_convert_type(v, jnp.int32)
                idx = jnp.bitwise_and(lax.shift_right_logical(bits, shift),
                                      _NB - 1)
                if prefix is None:
                    plsc.addupdate_scatter(hist_c, [idx], ones16)
                    plsc.addupdate_scatter(hist_s, [idx], v)
                else:
                    m = lax.shift_right_logical(bits, prefix_shift) == prefix
                    plsc.addupdate_scatter(hist_c, [idx], ones16, mask=m)
                    plsc.addupdate_scatter(hist_s, [idx], v, mask=m)

        hist_pass(20, None, None)
        b1, c1, s1 = _scan_hist(hist_c, hist_s, _K)
        r2 = _K - c1

        hist_pass(10, 20, b1)
        b2, c2, s2 = _scan_hist(hist_c, hist_s, r2)
        r3 = r2 - c2

        pref2 = jnp.bitwise_or(lax.shift_left(b1, 10), b2)
        hist_pass(0, 10, pref2)
        b3, c3, s3 = _scan_hist(hist_c, hist_s, r3)
        rfinal = r3 - c3

        tbits = jnp.bitwise_or(lax.shift_left(pref2, 10), b3)
        tvec = lax.bitcast_convert_type(jnp.full((16,), tbits, jnp.int32),
                                        jnp.float32)
        tval = jnp.max(tvec)
        sum_gt = s1 + s2 + s3
        anomaly = (sum_gt + rfinal.astype(jnp.float32) * tval) * (1.0 / _K)
        outvec = jnp.where(iota == j, anomaly, outvec)

        if j + 1 < _SC_RPW:
            nxt.wait()

    out_v[...] = outvec
    pltpu.sync_copy(out_v, out_hbm.at[wid])


def _lane_fold(acc):
    w = acc.shape[1]
    while w > 128:
        w //= 2
        acc = acc[:, :w] + acc[:, w:]
    return jnp.sum(acc, axis=-1, keepdims=True)


def _tc_topk_body(frame_ref, anom_ref):
    x = frame_ref[...]
    xi = lax.bitcast_convert_type(x, jnp.int32)

    lo = jnp.zeros((_TC_ROWS, 1), jnp.int32)
    hi = jnp.full((_TC_ROWS, 1), _HI_BITS, jnp.int32)

    def step(_, carry):
        lo, hi = carry
        mid = (lo + hi) >> 1
        acc = jnp.zeros((_TC_ROWS, _W_ACC), jnp.int32)
        for j in range(_T // _W_ACC):
            sl = xi[:, j * _W_ACC:(j + 1) * _W_ACC]
            acc = acc + (sl >= mid).astype(jnp.int32)
        cnt = _lane_fold(acc)
        pred = cnt >= _K
        return jnp.where(pred, mid, lo), jnp.where(pred, hi, mid)

    lo, hi = lax.fori_loop(0, _SEARCH_ITERS, step, (lo, hi))
    thr_bits = lo
    thr = lax.bitcast_convert_type(thr_bits, jnp.float32)

    acc_c = jnp.zeros((_TC_ROWS, _W_ACC), jnp.float32)
    acc_s = jnp.zeros((_TC_ROWS, _W_ACC), jnp.float32)
    for j in range(_T // _W_ACC):
        sl_i = xi[:, j * _W_ACC:(j + 1) * _W_ACC]
        sl_x = x[:, j * _W_ACC:(j + 1) * _W_ACC]
        gt = sl_i > thr_bits
        acc_c = acc_c + gt.astype(jnp.float32)
        acc_s = acc_s + jnp.where(gt, sl_x, 0.0)
    cnt_gt = _lane_fold(acc_c)
    sum_gt = _lane_fold(acc_s)
    anom_ref[...] = (sum_gt + (_K - cnt_gt) * thr) * (1.0 / _K)


def _bce_body(anom_ref, label_ref, out_ref):
    a = anom_ref[...]
    lab = label_ref[...]
    logp = jnp.maximum(jnp.log(a), -100.0)
    log1mp = jnp.maximum(jnp.log(1.0 - a), -100.0)
    total = jnp.sum(lab * logp + (1.0 - lab) * log1mp)
    out_ref[...] = jnp.full((1, 1), total * (-1.0 / _ROWS), jnp.float32)


@jax.jit
def kernel(frame, _label):
    anom_sc2 = _sc_topk(frame)
    anom_tc = pl.pallas_call(
        _tc_topk_body,
        grid=(1,),
        in_specs=[pl.BlockSpec((_TC_ROWS, _T), lambda i: (0, 0))],
        out_specs=pl.BlockSpec((_TC_ROWS, 1), lambda i: (0, 0)),
        out_shape=jax.ShapeDtypeStruct((_TC_ROWS, 1), jnp.float32),
    )(frame)
    anom_sc = anom_sc2[:, :_SC_RPW].reshape(_SC_ROWS, 1)
    anom = jnp.concatenate([anom_tc, anom_sc], axis=0)
    label = _label.astype(jnp.float32).reshape(_ROWS, 1)
    out = pl.pallas_call(
        _bce_body,
        out_shape=jax.ShapeDtypeStruct((1, 1), jnp.float32),
    )(anom, label)
    return out[0, 0]


# speedup vs baseline: 1.2575x; 1.2575x over previous
import functools

import jax
import jax.numpy as jnp
from jax import lax
from jax.experimental import pallas as pl
from jax.experimental.pallas import tpu as pltpu
from jax.experimental.pallas import tpu_sc as plsc

_T = 32768
_K = _T // 16 + 1
_ROWS = 128
_NV = _T // 16
_NB = 1024
_NW = 32

_TC_ROWS = 64
_SC_ROWS = _ROWS - _TC_ROWS
_SC_RPW = _SC_ROWS // _NW

_HI_BITS = 0x3F800000
_SEARCH_ITERS = 30
_W_ACC = 2048

_mesh = plsc.VectorSubcoreMesh(core_axis_name="c", subcore_axis_name="s")


def _scan_hist(hist_c, r):
    iota = lax.iota(jnp.int32, 16)

    def coarse(i, carry):
        c_hi, jsel, cbase = carry
        j = (_NB // 16 - 1) - i
        tot = jnp.sum(hist_c[pl.ds(j * 16, 16)])
        new_c = c_hi + tot
        hit = jnp.logical_and(c_hi < r, new_c >= r)
        jsel = jnp.where(hit, j, jsel)
        cbase = jnp.where(hit, c_hi, cbase)
        return new_c, jsel, cbase

    _, jsel, cbase = lax.fori_loop(
        0, _NB // 16, coarse, (jnp.int32(0), jnp.int32(0), jnp.int32(0)))

    c = hist_c[pl.ds(jsel * 16, 16)]
    cs = jnp.cumsum(c)
    tot = jnp.max(cs)
    suf_next = cbase + tot - cs
    cond = jnp.logical_and(suf_next + c >= r, suf_next < r)
    bsel = jnp.max(jnp.where(cond, jsel * 16 + iota, -1))
    c_above = jnp.max(jnp.where(cond, suf_next, -1))
    return bsel, c_above


@functools.partial(
    pl.kernel,
    mesh=_mesh,
    out_type=jax.ShapeDtypeStruct((_NW, 16), jnp.float32),
    compiler_params=pltpu.CompilerParams(needs_layout_passes=False),
    scratch_types=[
        pltpu.VMEM((_T,), jnp.float32),
        pltpu.VMEM((_T,), jnp.float32),
        pltpu.VMEM((_NB,), jnp.int32),
        pltpu.VMEM((16,), jnp.float32),
        pltpu.SemaphoreType.DMA,
    ],
)
def _sc_topk(frame_hbm, out_hbm, row0_v, row1_v, hist_c, out_v, sem):
    wid = lax.axis_index("s") * 2 + lax.axis_index("c")
    ones16 = jnp.ones((16,), jnp.int32)
    iota = lax.iota(jnp.int32, 16)
    outvec = jnp.zeros((16,), jnp.float32)
    bufs = (row0_v, row1_v)
    base = _TC_ROWS + wid * _SC_RPW

    pltpu.async_copy(frame_hbm.at[base], bufs[0], sem).wait()

    for j in range(_SC_RPW):
        row_v = bufs[j % 2]
        if j + 1 < _SC_RPW:
            nxt = pltpu.async_copy(frame_hbm.at[base + j + 1],
                                   bufs[(j + 1) % 2], sem)

        def hist_pass(shift, prefix_shift, prefix):
            z_i = jnp.zeros((16,), jnp.int32)

            @plsc.parallel_loop(0, _NB // 16, 1, unroll=8)
            def _(i):
                hist_c[pl.ds(i * 16, 16)] = z_i

            @plsc.parallel_loop(0, _NV, 1, unroll=8)
            def _(i):
                v = row_v[pl.ds(i * 16, 16)]
                bits = lax.bitcast_convert_type(v, jnp.int32)
                idx = jnp.bitwise_and(lax.shift_right_logical(bits, shift),
                                      _NB - 1)
                if prefix is None:
                    plsc.addupdate_scatter(hist_c, [idx], ones16)
                else:
                    m = lax.shift_right_logical(bits, prefix_shift) == prefix
                    plsc.addupdate_scatter(hist_c, [idx], ones16, mask=m)

        hist_pass(20, None, None)
        b1, c1 = _scan_hist(hist_c, _K)
        r2 = _K - c1

        hist_pass(10, 20, b1)
        b2, c2 = _scan_hist(hist_c, r2)
        r3 = r2 - c2

        pref2 = jnp.bitwise_or(lax.shift_left(b1, 10), b2)
        hist_pass(0, 10, pref2)
        b3, c3 = _scan_hist(hist_c, r3)
        rfinal = r3 - c3
        tbits = jnp.bitwise_or(lax.shift_left(pref2, 10), b3)

        def fsum(i, a):
            v = row_v[pl.ds(i * 16, 16)]
            bits = lax.bitcast_convert_type(v, jnp.int32)
            return a + jnp.where(bits > tbits, v, 0.0)

        acc = plsc.parallel_loop(0, _NV, 1, unroll=8,
                                 carry=jnp.zeros((16,), jnp.float32))(fsum)
        sum_gt = jnp.sum(acc)

        tvec = lax.bitcast_convert_type(jnp.full((16,), tbits, jnp.int32),
                                        jnp.float32)
        tval = jnp.max(tvec)
        anomaly = (sum_gt + rfinal.astype(jnp.float32) * tval) * (1.0 / _K)
        outvec = jnp.where(iota == j, anomaly, outvec)

        if j + 1 < _SC_RPW:
            nxt.wait()

    out_v[...] = outvec
    pltpu.sync_copy(out_v, out_hbm.at[wid])


def _lane_fold(acc):
    w = acc.shape[1]
    while w > 128:
        w //= 2
        acc = acc[:, :w] + acc[:, w:]
    return jnp.sum(acc, axis=-1, keepdims=True)


def _tc_topk_body(frame_ref, anom_ref):
    x = frame_ref[...]
    xi = lax.bitcast_convert_type(x, jnp.int32)

    lo = jnp.zeros((_TC_ROWS, 1), jnp.int32)
    hi = jnp.full((_TC_ROWS, 1), _HI_BITS, jnp.int32)

    def step(_, carry):
        lo, hi = carry
        mid = (lo + hi) >> 1
        acc = jnp.zeros((_TC_ROWS, _W_ACC), jnp.int32)
        for j in range(_T // _W_ACC):
            sl = xi[:, j * _W_ACC:(j + 1) * _W_ACC]
            acc = acc + (sl >= mid).astype(jnp.int32)
        cnt = _lane_fold(acc)
        pred = cnt >= _K
        return jnp.where(pred, mid, lo), jnp.where(pred, hi, mid)

    lo, hi = lax.fori_loop(0, _SEARCH_ITERS, step, (lo, hi))
    thr_bits = lo
    thr = lax.bitcast_convert_type(thr_bits, jnp.float32)

    acc_c = jnp.zeros((_TC_ROWS, _W_ACC), jnp.float32)
    acc_s = jnp.zeros((_TC_ROWS, _W_ACC), jnp.float32)
    for j in range(_T // _W_ACC):
        sl_i = xi[:, j * _W_ACC:(j + 1) * _W_ACC]
        sl_x = x[:, j * _W_ACC:(j + 1) * _W_ACC]
        gt = sl_i > thr_bits
        acc_c = acc_c + gt.astype(jnp.float32)
        acc_s = acc_s + jnp.where(gt, sl_x, 0.0)
    cnt_gt = _lane_fold(acc_c)
    sum_gt = _lane_fold(acc_s)
    anom_ref[...] = (sum_gt + (_K - cnt_gt) * thr) * (1.0 / _K)


def _bce_body(anom_ref, label_ref, out_ref):
    a = anom_ref[...]
    lab = label_ref[...]
    logp = jnp.maximum(jnp.log(a), -100.0)
    log1mp = jnp.maximum(jnp.log(1.0 - a), -100.0)
    total = jnp.sum(lab * logp + (1.0 - lab) * log1mp)
    out_ref[...] = jnp.full((1, 1), total * (-1.0 / _ROWS), jnp.float32)


@jax.jit
def kernel(frame, _label):
    anom_sc2 = _sc_topk(frame)
    anom_tc = pl.pallas_call(
        _tc_topk_body,
        grid=(1,),
        in_specs=[pl.BlockSpec((_TC_ROWS, _T), lambda i: (0, 0))],
        out_specs=pl.BlockSpec((_TC_ROWS, 1), lambda i: (0, 0)),
        out_shape=jax.ShapeDtypeStruct((_TC_ROWS, 1), jnp.float32),
    )(frame)
    anom_sc = anom_sc2[:, :_SC_RPW].reshape(_SC_ROWS, 1)
    anom = jnp.concatenate([anom_tc, anom_sc], axis=0)
    label = _label.astype(jnp.float32).reshape(_ROWS, 1)
    out = pl.pallas_call(
        _bce_body,
        out_shape=jax.ShapeDtypeStruct((1, 1), jnp.float32),
    )(anom, label)
    return out[0, 0]
